# baseline (speedup 1.0000x reference)
import functools

import jax
import jax.numpy as jnp
from jax import lax
from jax.experimental import pallas as pl
from jax.experimental.pallas import tpu as pltpu
from jax.experimental.pallas import tpu_sc as plsc

TOKENS = 16384
HID = 2048
NEXP = 64
K = 8
TM = 2048

NC = 2
NS = 16
NW = NC * NS
RPW = TOKENS // NW


def _scores_body(x_ref, w_ref, s_ref):
    logits = jax.lax.dot_general(
        x_ref[...], w_ref[...], (((1,), (1,)), ((), ())),
        preferred_element_type=jnp.float32,
    )
    s_ref[...] = jax.nn.sigmoid(logits)


def _tc_scores(x, W):
    return pl.pallas_call(
        _scores_body,
        grid=(TOKENS // TM,),
        in_specs=[
            pl.BlockSpec((TM, HID), lambda i: (i, 0)),
            pl.BlockSpec((NEXP, HID), lambda i: (0, 0)),
        ],
        out_specs=pl.BlockSpec((TM, NEXP), lambda i: (i, 0)),
        out_shape=jax.ShapeDtypeStruct((TOKENS, NEXP), jnp.float32),
        compiler_params=pltpu.CompilerParams(
            dimension_semantics=("parallel",),
        ),
    )(x, W)


_sc_mesh = plsc.VectorSubcoreMesh(core_axis_name="c", subcore_axis_name="s")


@functools.partial(
    pl.kernel,
    mesh=_sc_mesh,
    out_type=[
        jax.ShapeDtypeStruct((TOKENS * K,), jnp.int32),
        jax.ShapeDtypeStruct((TOKENS * K,), jnp.float32),
    ],
    scratch_types=[
        pltpu.VMEM((RPW * NEXP,), jnp.float32),
        pltpu.VMEM((NEXP,), jnp.float32),
        pltpu.VMEM((RPW * K + 8,), jnp.int32),
        pltpu.VMEM((RPW * K + 8,), jnp.float32),
    ],
    compiler_params=pltpu.CompilerParams(needs_layout_passes=False),
)
def _sc_topk(scores_hbm, bias_hbm, idx_hbm, wgt_hbm, sv, bv, oi, ow):
    wid = lax.axis_index("s") * NC + lax.axis_index("c")
    base = wid * RPW
    pltpu.sync_copy(scores_hbm.at[pl.ds(base * NEXP, RPW * NEXP)], sv)
    pltpu.sync_copy(bias_hbm, bv)

    lane = lax.iota(jnp.int32, 16)
    low8 = lane < 8
    bias_q = [bv[pl.ds(q * 16, 16)] for q in range(4)]

    def merge(ka, va, kb, vb, descending):
        kc = jnp.where(low8, ka, kb)
        vc = jnp.where(low8, va, vb)
        return plsc.sort_key_val(kc, vc, descending=descending)

    def body(t, carry):
        off = t * NEXP
        ks, vs = [], []
        for q in range(4):
            s_q = sv[pl.ds(off + q * 16, 16)]
            k_q = s_q + bias_q[q]
            v_q = lane + q * 16
            k8, v8 = plsc.sort_key_val(k_q, v_q, descending=(q % 2 == 0))
            ks.append(k8)
            vs.append(v8)
        k01, v01 = merge(ks[0], vs[0], ks[1], vs[1], True)
        k23, v23 = merge(ks[2], vs[2], ks[3], vs[3], False)
        _, vf = merge(k01, v01, k23, v23, True)
        uw = plsc.load_gather(sv, [off + vf])
        ssum = jnp.sum(jnp.where(low8, uw, 0.0))
        wn = uw / (ssum + 1e-20)
        plsc.store_compressed(oi.at[pl.ds(t * K, 16)], vf, mask=low8)
        plsc.store_compressed(ow.at[pl.ds(t * K, 16)], wn, mask=low8)
        return carry

    lax.fori_loop(0, RPW, body, 0)
    pltpu.sync_copy(oi.at[pl.ds(0, RPW * K)], idx_hbm.at[pl.ds(base * K, RPW * K)])
    pltpu.sync_copy(ow.at[pl.ds(0, RPW * K)], wgt_hbm.at[pl.ds(base * K, RPW * K)])


@jax.jit
def kernel(x, W, e_score_correction_bias):
    scores = _tc_scores(x, W)
    idx_flat, wgt_flat = _sc_topk(scores.reshape(-1), e_score_correction_bias)
    return (idx_flat.reshape(TOKENS, K), wgt_flat.reshape(TOKENS, K))


# speedup vs baseline: 2.0577x; 2.0577x over previous
import functools

import jax
import jax.numpy as jnp
from jax.experimental import pallas as pl
from jax.experimental.pallas import tpu as pltpu

TOKENS = 16384
HID = 2048
NEXP = 64
K = 8
TM = 2048


def _gate_body(x_ref, w_ref, b_ref, idx_ref, wgt_ref):
    x = x_ref[...]
    w = w_ref[...]
    logits = jax.lax.dot_general(
        w, x, (((1,), (1,)), ((), ())), preferred_element_type=jnp.float32
    )
    scores = jax.nn.sigmoid(logits)
    biased = scores + b_ref[...]
    iota = jax.lax.broadcasted_iota(jnp.int32, (NEXP, TM), 0)
    idxs, vals = [], []
    cur = biased
    for _ in range(K):
        m = jnp.max(cur, axis=0, keepdims=True)
        cand = jnp.where(cur == m, iota, NEXP)
        idx = jnp.min(cand, axis=0, keepdims=True)
        sel = cand == idx
        sval = jnp.sum(jnp.where(sel, scores, 0.0), axis=0, keepdims=True)
        cur = jnp.where(sel, -jnp.inf, cur)
        idxs.append(idx)
        vals.append(sval)
    topk_i = jnp.concatenate(idxs, axis=0)
    topk_v = jnp.concatenate(vals, axis=0)
    s = jnp.sum(topk_v, axis=0, keepdims=True) + 1e-20
    idx_ref[...] = topk_i.T
    wgt_ref[...] = (topk_v / s).T


@jax.jit
def kernel(x, W, e_score_correction_bias):
    bias2d = e_score_correction_bias.reshape(NEXP, 1)
    grid = (TOKENS // TM,)
    out_i, out_w = pl.pallas_call(
        _gate_body,
        grid=grid,
        in_specs=[
            pl.BlockSpec((TM, HID), lambda i: (i, 0)),
            pl.BlockSpec((NEXP, HID), lambda i: (0, 0)),
            pl.BlockSpec((NEXP, 1), lambda i: (0, 0)),
        ],
        out_specs=[
            pl.BlockSpec((TM, K), lambda i: (i, 0)),
            pl.BlockSpec((TM, K), lambda i: (i, 0)),
        ],
        out_shape=[
            jax.ShapeDtypeStruct((TOKENS, K), jnp.int32),
            jax.ShapeDtypeStruct((TOKENS, K), jnp.float32),
        ],
        compiler_params=pltpu.CompilerParams(
            dimension_semantics=("parallel",),
        ),
    )(x, W, bias2d)
    return (out_i, out_w)
